# baseline (speedup 1.0000x reference)
import functools

import jax
import jax.numpy as jnp
from jax import lax
from jax.experimental import pallas as pl
from jax.experimental.pallas import tpu as pltpu
from jax.experimental.pallas import tpu_sc as plsc

BATCH = 16384
EMB_DIM = 32
NUM_F = 4
LANES = 16
NUM_CORES = 2
NUM_SUBCORES = 16
NW = NUM_CORES * NUM_SUBCORES
BPW = BATCH // NW


def _build():
    mesh = plsc.VectorSubcoreMesh(core_axis_name="c", subcore_axis_name="s")

    @functools.partial(
        pl.kernel,
        mesh=mesh,
        compiler_params=pltpu.CompilerParams(use_tc_tiling_on_sc=False),
        out_type=jax.ShapeDtypeStruct((BATCH, EMB_DIM), jnp.float32),
        scratch_types=[
            pltpu.VMEM((BPW,), jnp.int32),
            pltpu.VMEM((BPW,), jnp.int32),
            pltpu.VMEM((BPW,), jnp.int32),
            pltpu.VMEM((BPW,), jnp.int32),
            pltpu.VMEM((BPW, EMB_DIM), jnp.float32),
            pltpu.VMEM((BPW, EMB_DIM), jnp.float32),
            pltpu.VMEM((BPW, EMB_DIM), jnp.float32),
            pltpu.VMEM((BPW, EMB_DIM), jnp.float32),
            pltpu.SemaphoreType.DMA,
        ],
    )
    def eges(idxs_hbm, t0, t1, t2, t3, out_hbm,
             i0, i1, i2, i3, r0, r1, r2, r3, sem):
        wid = lax.axis_index("s") * NUM_CORES + lax.axis_index("c")
        base = wid * BPW
        pltpu.sync_copy(idxs_hbm.at[0, pl.ds(base, BPW)], i0)
        pltpu.sync_copy(idxs_hbm.at[1, pl.ds(base, BPW)], i1)
        pltpu.sync_copy(idxs_hbm.at[2, pl.ds(base, BPW)], i2)
        pltpu.sync_copy(idxs_hbm.at[3, pl.ds(base, BPW)], i3)
        c0 = pltpu.async_copy(t0.at[i0], r0, sem)
        c1 = pltpu.async_copy(t1.at[i1], r1, sem)
        c2 = pltpu.async_copy(t2.at[i2], r2, sem)
        c3 = pltpu.async_copy(t3.at[i3], r3, sem)
        c0.wait()
        c1.wait()
        c2.wait()
        c3.wait()

        def body(j, carry):
            for k in range(EMB_DIM // LANES):
                sl = pl.ds(k * LANES, LANES)
                r0[j, sl] = r0[j, sl] + r1[j, sl] + r2[j, sl] + r3[j, sl]
            return carry

        lax.fori_loop(0, BPW, body, 0)
        pltpu.sync_copy(r0, out_hbm.at[pl.ds(base, BPW)])

    return eges


_EGES = _build()


def kernel(feature, label, table0, table1, table2, table3, node_table):
    del label, node_table
    idxs = feature.T
    return _EGES(idxs, table0, table1, table2, table3)
